# Initial kernel scaffold; baseline (speedup 1.0000x reference)
import functools

import jax
import jax.numpy as jnp
from jax import lax
from jax.experimental import pallas as pl
from jax.experimental.pallas import tpu as pltpu
from jax.experimental.pallas import tpu_sc as plsc

_NB = 256
_CLIP = 40.0
_TS = 64
_H = 512
_IMGS = 24
_NTILES = _IMGS * 8 * 8
_NC = 2
_NS = 16
_NW = _NC * _NS
_HIST_STRIPS_PER_W = (_IMGS * 8) // _NW
_APPLY_STRIPS_PER_W = (_IMGS * 16) // _NW


def _minmax_body(x_ref, mn_ref, mx_ref):
    i = pl.program_id(0)
    xb = x_ref[...]
    m = jnp.min(xb)
    mx = jnp.max(xb)

    @pl.when(i == 0)
    def _():
        mn_ref[0, 0] = m
        mx_ref[0, 0] = mx

    @pl.when(i > 0)
    def _():
        mn_ref[0, 0] = jnp.minimum(mn_ref[0, 0], m)
        mx_ref[0, 0] = jnp.maximum(mx_ref[0, 0], mx)


def _minmax(x):
    return pl.pallas_call(
        _minmax_body,
        grid=(_IMGS,),
        in_specs=[pl.BlockSpec((1, _H, _H), lambda i: (i, 0, 0))],
        out_specs=[pl.BlockSpec((1, 1), lambda i: (0, 0)),
                   pl.BlockSpec((1, 1), lambda i: (0, 0))],
        out_shape=[jax.ShapeDtypeStruct((1, 1), jnp.float32),
                   jax.ShapeDtypeStruct((1, 1), jnp.float32)],
    )(x)


def _histlut_body(x_hbm, prm_hbm, lut_hbm, xs_v, h16_v, h256_v, prm_v):
    wid = lax.axis_index("s") * _NC + lax.axis_index("c")
    pltpu.sync_copy(prm_hbm, prm_v)
    mn = prm_v[0, :]
    sc = prm_v[1, :]
    lanes = lax.iota(jnp.int32, 16)
    zero16 = jnp.zeros((16,), jnp.float32)
    ones16 = jnp.ones((16,), jnp.float32)

    def strip_body(sidx, _):
        strip = wid * _HIST_STRIPS_PER_W + sidx
        img = strip // 8
        ty = strip % 8
        pltpu.sync_copy(x_hbm.at[img, pl.ds(ty * _TS, _TS)], xs_v)

        def tile_body(gx, _):
            def zrow(l, _):
                def zgrp(g, _):
                    h16_v[l, pl.ds(g * 16, 16)] = zero16
                    return 0
                return lax.fori_loop(0, 16, zgrp, 0)
            lax.fori_loop(0, 16, zrow, 0)

            def row_body(r, _):
                def grp_body(j, _):
                    col = gx * _TS + j * 16
                    xv = xs_v[r, pl.ds(col, 16)]
                    vf = (xv - mn) * sc + 0.5
                    vi = jnp.minimum(vf.astype(jnp.int32), _NB - 1)
                    plsc.addupdate_scatter(h16_v, [lanes, vi], ones16)
                    return 0
                return lax.fori_loop(0, 4, grp_body, 0)
            lax.fori_loop(0, _TS, row_body, 0)

            def red_body(g, _):
                def racc(l, a):
                    return a + h16_v[l, pl.ds(g * 16, 16)]
                acc = lax.fori_loop(0, 16, racc, zero16)
                h256_v[pl.ds(g * 16, 16)] = acc
                return 0
            lax.fori_loop(0, 16, red_body, 0)

            def exc_body(g, tot):
                h = h256_v[pl.ds(g * 16, 16)]
                return tot + jnp.sum(h - jnp.minimum(h, _CLIP))
            excess = lax.fori_loop(0, 16, exc_body, jnp.float32(0.0))
            add = excess * (1.0 / _NB)

            def lut_body(g, carry):
                h = jnp.minimum(h256_v[pl.ds(g * 16, 16)], _CLIP) + add
                cs = jnp.cumsum(h) + carry
                lutv = jnp.minimum(jnp.maximum(cs * (1.0 / 4096.0), 0.0), 1.0)
                h256_v[pl.ds(g * 16, 16)] = lutv
                return carry + jnp.sum(h)
            lax.fori_loop(0, 16, lut_body, jnp.float32(0.0))

            tile = strip * 8 + gx
            pltpu.sync_copy(h256_v, lut_hbm.at[tile])
            return 0
        lax.fori_loop(0, 8, tile_body, 0)
        return 0
    lax.fori_loop(0, _HIST_STRIPS_PER_W, strip_body, 0)


def _apply_body(x_hbm, lut_hbm, prm_hbm, out_hbm,
                xs_v, os_v, lutA_v, lutB_v, wy_v, prm_v):
    wid = lax.axis_index("s") * _NC + lax.axis_index("c")
    pltpu.sync_copy(prm_hbm, prm_v)
    mn = prm_v[0, :]
    sc = prm_v[1, :]
    iotaf = lax.iota(jnp.int32, 16).astype(jnp.float32)

    def strip_body(sidx, _):
        strip = wid * _APPLY_STRIPS_PER_W + sidx
        img = strip // 16
        qy = strip % 16
        ty = qy // 2
        y0 = jnp.where(qy % 2 == 0, jnp.maximum(ty - 1, 0), ty)
        y1 = jnp.where(qy % 2 == 0, ty, jnp.minimum(ty + 1, 7))
        pltpu.sync_copy(x_hbm.at[img, pl.ds(qy * 32, 32)], xs_v)
        pltpu.sync_copy(lut_hbm.at[img, y0], lutA_v)
        pltpu.sync_copy(lut_hbm.at[img, y1], lutB_v)

        for k in (0, 1):
            rowf = iotaf + (qy * 32 + k * 16).astype(jnp.float32)
            z = (rowf + 0.5) * (1.0 / 64.0) - 0.5
            zi = (z + 8.0).astype(jnp.int32)
            wy_v[pl.ds(k * 16, 16)] = z - (zi.astype(jnp.float32) - 8.0)

        def qx_body(qx, _):
            tx = qx // 2
            x0 = jnp.where(qx % 2 == 0, jnp.maximum(tx - 1, 0), tx)
            x1 = jnp.where(qx % 2 == 0, tx, jnp.minimum(tx + 1, 7))
            x0v = jnp.broadcast_to(x0, (16,))
            x1v = jnp.broadcast_to(x1, (16,))
            colf0 = iotaf + (qx * 32).astype(jnp.float32)
            wxs = []
            for j in (0, 1):
                zc = (colf0 + (j * 16 + 0.5)) * (1.0 / 64.0) - 0.5
                zci = (zc + 8.0).astype(jnp.int32)
                wxs.append(zc - (zci.astype(jnp.float32) - 8.0))

            def row_body(r, _):
                wyv = jnp.broadcast_to(wy_v[r], (16,))
                for j in (0, 1):
                    col = qx * 32 + j * 16
                    xv = xs_v[r, pl.ds(col, 16)]
                    vf = (xv - mn) * sc + 0.5
                    vi = jnp.minimum(vf.astype(jnp.int32), _NB - 1)
                    g00 = plsc.load_gather(lutA_v, [x0v, vi])
                    g01 = plsc.load_gather(lutA_v, [x1v, vi])
                    g10 = plsc.load_gather(lutB_v, [x0v, vi])
                    g11 = plsc.load_gather(lutB_v, [x1v, vi])
                    wx = wxs[j]
                    top = g00 + wx * (g01 - g00)
                    bot = g10 + wx * (g11 - g10)
                    os_v[r, pl.ds(col, 16)] = top + wyv * (bot - top)
                return 0
            lax.fori_loop(0, 32, row_body, 0)
            return 0
        lax.fori_loop(0, 16, qx_body, 0)
        pltpu.sync_copy(os_v, out_hbm.at[img, pl.ds(qy * 32, 32)])
        return 0
    lax.fori_loop(0, _APPLY_STRIPS_PER_W, strip_body, 0)


_SC_MESH = plsc.VectorSubcoreMesh(core_axis_name="c", subcore_axis_name="s")

_histlut = functools.partial(
    pl.kernel,
    mesh=_SC_MESH,
    out_type=jax.ShapeDtypeStruct((_NTILES, _NB), jnp.float32),
    scratch_types=[
        pltpu.VMEM((_TS, _H), jnp.float32),
        pltpu.VMEM((16, _NB), jnp.float32),
        pltpu.VMEM((_NB,), jnp.float32),
        pltpu.VMEM((2, 16), jnp.float32),
    ],
)(_histlut_body)

_apply = functools.partial(
    pl.kernel,
    mesh=_SC_MESH,
    out_type=jax.ShapeDtypeStruct((_IMGS, _H, _H), jnp.float32),
    scratch_types=[
        pltpu.VMEM((32, _H), jnp.float32),
        pltpu.VMEM((32, _H), jnp.float32),
        pltpu.VMEM((8, _NB), jnp.float32),
        pltpu.VMEM((8, _NB), jnp.float32),
        pltpu.VMEM((32,), jnp.float32),
        pltpu.VMEM((2, 16), jnp.float32),
    ],
)(_apply_body)


def kernel(t):
    x = t[0].reshape(_IMGS, _H, _H)
    mn_a, mx_a = _minmax(x)
    mn = mn_a[0, 0]
    scale = 255.0 / (mx_a[0, 0] - mn + 1e-12)
    prm = jnp.stack([jnp.broadcast_to(mn, (16,)),
                     jnp.broadcast_to(scale, (16,))])
    lut = _histlut(x, prm)
    out = _apply(x, lut.reshape(_IMGS, 8, 8, _NB), prm)
    out0 = out.reshape(t.shape[1:])
    return (out0,) + tuple(t[i] for i in range(1, t.shape[0]))


# speedup vs baseline: 474.0298x; 474.0298x over previous
import functools

import jax
import jax.numpy as jnp
from jax import lax
from jax.experimental import pallas as pl
from jax.experimental.pallas import tpu as pltpu
from jax.experimental.pallas import tpu_sc as plsc

_NB = 256
_CLIP = 40.0
_TS = 64
_H = 512
_IMGS = 24
_NTILES = _IMGS * 8 * 8
_NC = 2
_NS = 16
_NW = _NC * _NS
_HIST_STRIPS_PER_W = (_IMGS * 8) // _NW
_APPLY_STRIPS_PER_W = (_IMGS * 16) // _NW


def _minmax_body(x_ref, mn_ref, mx_ref):
    i = pl.program_id(0)
    xb = x_ref[...]
    m = jnp.min(xb)
    mx = jnp.max(xb)

    @pl.when(i == 0)
    def _():
        mn_ref[0, 0] = m
        mx_ref[0, 0] = mx

    @pl.when(i > 0)
    def _():
        mn_ref[0, 0] = jnp.minimum(mn_ref[0, 0], m)
        mx_ref[0, 0] = jnp.maximum(mx_ref[0, 0], mx)


def _minmax(x):
    return pl.pallas_call(
        _minmax_body,
        grid=(_IMGS,),
        in_specs=[pl.BlockSpec((1, _H, _H), lambda i: (i, 0, 0))],
        out_specs=[pl.BlockSpec((1, 1), lambda i: (0, 0),
                                memory_space=pltpu.SMEM),
                   pl.BlockSpec((1, 1), lambda i: (0, 0),
                                memory_space=pltpu.SMEM)],
        out_shape=[jax.ShapeDtypeStruct((1, 1), jnp.float32),
                   jax.ShapeDtypeStruct((1, 1), jnp.float32)],
    )(x)


def _histlut_body(x_hbm, prm_hbm, lut_hbm, xs_v, h16_v, h256_v, prm_v):
    wid = lax.axis_index("s") * _NC + lax.axis_index("c")
    pltpu.sync_copy(prm_hbm, prm_v)
    mn = prm_v[0, :]
    sc = prm_v[1, :]
    lanes256 = lax.iota(jnp.int32, 16) * _NB
    zero16 = jnp.zeros((16,), jnp.float32)
    ones16 = jnp.ones((16,), jnp.float32)

    def strip_body(sidx, _):
        strip = wid * _HIST_STRIPS_PER_W + sidx
        img = strip // 8
        ty = strip % 8
        pltpu.sync_copy(x_hbm.at[img, pl.ds(ty * _TS, _TS)], xs_v)

        def tile_body(gx, _):
            def zgrp(g, _):
                h16_v[pl.ds(g * 16, 16)] = zero16
                return 0
            lax.fori_loop(0, 256, zgrp, 0)

            def row_body(r, _):
                def grp_body(j, _):
                    col = gx * _TS + j * 16
                    xv = xs_v[r, pl.ds(col, 16)]
                    vf = (xv - mn) * sc + 0.5
                    vi = jnp.minimum(vf.astype(jnp.int32), _NB - 1)
                    plsc.addupdate_scatter(h16_v, [lanes256 + vi], ones16)
                    return 0
                return lax.fori_loop(0, 4, grp_body, 0)
            lax.fori_loop(0, _TS, row_body, 0)

            def red_body(g, _):
                def racc(l, a):
                    return a + h16_v[pl.ds(l * _NB + g * 16, 16)]
                acc = lax.fori_loop(0, 16, racc, zero16)
                h256_v[pl.ds(g * 16, 16)] = acc
                return 0
            lax.fori_loop(0, 16, red_body, 0)

            def exc_body(g, tot):
                h = h256_v[pl.ds(g * 16, 16)]
                return tot + jnp.sum(h - jnp.minimum(h, _CLIP))
            excess = lax.fori_loop(0, 16, exc_body, jnp.float32(0.0))
            add = excess * (1.0 / _NB)

            def lut_body(g, carry):
                h = jnp.minimum(h256_v[pl.ds(g * 16, 16)], _CLIP) + add
                cs = jnp.cumsum(h) + carry
                lutv = jnp.minimum(jnp.maximum(cs * (1.0 / 4096.0), 0.0), 1.0)
                h256_v[pl.ds(g * 16, 16)] = lutv
                return carry + jnp.sum(h)
            lax.fori_loop(0, 16, lut_body, jnp.float32(0.0))

            tile = strip * 8 + gx
            pltpu.sync_copy(h256_v, lut_hbm.at[tile])
            return 0
        lax.fori_loop(0, 8, tile_body, 0)
        return 0
    lax.fori_loop(0, _HIST_STRIPS_PER_W, strip_body, 0)


def _apply_body(x_hbm, lut_hbm, prm_hbm, out_hbm,
                xs_v, os_v, lutA_v, lutB_v, wy_v, prm_v):
    wid = lax.axis_index("s") * _NC + lax.axis_index("c")
    pltpu.sync_copy(prm_hbm, prm_v)
    mn = prm_v[0, :]
    sc = prm_v[1, :]
    iotaf = lax.iota(jnp.int32, 16).astype(jnp.float32)

    def strip_body(sidx, _):
        strip = wid * _APPLY_STRIPS_PER_W + sidx
        img = strip // 16
        qy = strip % 16
        ty = qy // 2
        y0 = jnp.where(qy % 2 == 0, jnp.maximum(ty - 1, 0), ty)
        y1 = jnp.where(qy % 2 == 0, ty, jnp.minimum(ty + 1, 7))
        pltpu.sync_copy(x_hbm.at[img, pl.ds(qy * 32, 32)], xs_v)
        pltpu.sync_copy(lut_hbm.at[img, y0], lutA_v)
        pltpu.sync_copy(lut_hbm.at[img, y1], lutB_v)

        for k in (0, 1):
            rowf = iotaf + (qy * 32 + k * 16).astype(jnp.float32)
            z = (rowf + 0.5) * (1.0 / 64.0) - 0.5
            zi = (z + 8.0).astype(jnp.int32)
            wy_v[pl.ds(k * 16, 16)] = z - (zi.astype(jnp.float32) - 8.0)

        def qx_body(qx, _):
            tx = qx // 2
            x0 = jnp.where(qx % 2 == 0, jnp.maximum(tx - 1, 0), tx)
            x1 = jnp.where(qx % 2 == 0, tx, jnp.minimum(tx + 1, 7))
            x0v = jnp.broadcast_to(x0 * _NB, (16,))
            x1v = jnp.broadcast_to(x1 * _NB, (16,))
            colf0 = iotaf + (qx * 32).astype(jnp.float32)
            wxs = []
            for j in (0, 1):
                zc = (colf0 + (j * 16 + 0.5)) * (1.0 / 64.0) - 0.5
                zci = (zc + 8.0).astype(jnp.int32)
                wxs.append(zc - (zci.astype(jnp.float32) - 8.0))

            def row_body(r, _):
                wyv = plsc.load_gather(wy_v, [jnp.broadcast_to(r, (16,))])
                for j in (0, 1):
                    col = qx * 32 + j * 16
                    xv = xs_v[r, pl.ds(col, 16)]
                    vf = (xv - mn) * sc + 0.5
                    vi = jnp.minimum(vf.astype(jnp.int32), _NB - 1)
                    g00 = plsc.load_gather(lutA_v, [x0v + vi])
                    g01 = plsc.load_gather(lutA_v, [x1v + vi])
                    g10 = plsc.load_gather(lutB_v, [x0v + vi])
                    g11 = plsc.load_gather(lutB_v, [x1v + vi])
                    wx = wxs[j]
                    top = g00 + wx * (g01 - g00)
                    bot = g10 + wx * (g11 - g10)
                    os_v[r, pl.ds(col, 16)] = top + wyv * (bot - top)
                return 0
            lax.fori_loop(0, 32, row_body, 0)
            return 0
        lax.fori_loop(0, 16, qx_body, 0)
        pltpu.sync_copy(os_v, out_hbm.at[img, pl.ds(qy * 32, 32)])
        return 0
    lax.fori_loop(0, _APPLY_STRIPS_PER_W, strip_body, 0)


_SC_MESH = plsc.VectorSubcoreMesh(core_axis_name="c", subcore_axis_name="s")
_SC_PARAMS = pltpu.CompilerParams(needs_layout_passes=False)

_histlut = functools.partial(
    pl.kernel,
    mesh=_SC_MESH,
    compiler_params=_SC_PARAMS,
    out_type=jax.ShapeDtypeStruct((_NTILES, _NB), jnp.float32),
    scratch_types=[
        pltpu.VMEM((_TS, _H), jnp.float32),
        pltpu.VMEM((16 * _NB,), jnp.float32),
        pltpu.VMEM((_NB,), jnp.float32),
        pltpu.VMEM((2, 16), jnp.float32),
    ],
)(_histlut_body)

_apply = functools.partial(
    pl.kernel,
    mesh=_SC_MESH,
    compiler_params=_SC_PARAMS,
    out_type=jax.ShapeDtypeStruct((_IMGS, _H, _H), jnp.float32),
    scratch_types=[
        pltpu.VMEM((32, _H), jnp.float32),
        pltpu.VMEM((32, _H), jnp.float32),
        pltpu.VMEM((8 * _NB,), jnp.float32),
        pltpu.VMEM((8 * _NB,), jnp.float32),
        pltpu.VMEM((32,), jnp.float32),
        pltpu.VMEM((2, 16), jnp.float32),
    ],
)(_apply_body)


def kernel(t):
    x = t[0].reshape(_IMGS, _H, _H)
    mn_a, mx_a = _minmax(x)
    mn = mn_a[0, 0]
    scale = 255.0 / (mx_a[0, 0] - mn + 1e-12)
    prm = jnp.stack([jnp.broadcast_to(mn, (16,)),
                     jnp.broadcast_to(scale, (16,))])
    lut = _histlut(x, prm)
    out = _apply(x, lut.reshape(_IMGS, 8, 8 * _NB), prm)
    out0 = out.reshape(t.shape[1:])
    return (out0,) + tuple(t[i] for i in range(1, t.shape[0]))
